# Initial kernel scaffold; baseline (speedup 1.0000x reference)
import jax
import jax.numpy as jnp
from jax.experimental import pallas as pl


def kernel(x, edge_index, W1, b1, Wmu, bmu, Wlog, blog):
    raise NotImplementedError("write your pallas kernel here")


# speedup vs baseline: 15.7709x; 15.7709x over previous
import functools

import jax
import jax.numpy as jnp
from jax import lax
from jax.experimental import pallas as pl
from jax.experimental.pallas import tpu as pltpu
from jax.experimental.pallas import tpu_sc as plsc

N = 10000
E = 320000
C = 128
OUT = 64
DEGW = 16

NC, NS = 2, 16
NW = NC * NS
EPW = E // NW
CHUNK = 80
NCH = EPW // CHUNK
ROWS_PT = 624
TAIL0 = NS * ROWS_PT
TAILN = N - TAIL0

_MESH = plsc.VectorSubcoreMesh(core_axis_name="c", subcore_axis_name="s")


def _deg_body(dst_hbm, ones_hbm, out_hbm, idx_v, ones_v, row_v, acc_sh):
    cid = lax.axis_index("c")
    sid = lax.axis_index("s")
    wid = sid * NC + cid
    r0 = sid * ROWS_PT
    pltpu.sync_copy(ones_hbm, ones_v)

    def zstep(i, carry):
        row_v[pl.ds(i * 16, 16)] = jnp.zeros((16,), jnp.float32)
        return carry

    lax.fori_loop(0, ROWS_PT // 16, zstep, 0)
    pltpu.sync_copy(row_v, acc_sh.at[pl.ds(r0, ROWS_PT)])

    @pl.when(sid == NS - 1)
    def _():
        pltpu.sync_copy(row_v.at[pl.ds(0, TAILN)], acc_sh.at[pl.ds(TAIL0, TAILN)])

    plsc.subcore_barrier()
    ebase = wid * EPW

    def step(c, carry):
        b = ebase + c * CHUNK
        pltpu.sync_copy(dst_hbm.at[pl.ds(b, CHUNK)], idx_v)
        pltpu.sync_copy(ones_v, acc_sh.at[idx_v], add=True)
        return carry

    lax.fori_loop(0, NCH, step, 0)
    plsc.subcore_barrier()
    obase = pl.multiple_of(cid * N + r0, 8)
    pltpu.sync_copy(acc_sh.at[pl.ds(r0, ROWS_PT)], row_v)
    pltpu.sync_copy(row_v, out_hbm.at[pl.ds(obase, ROWS_PT)])

    @pl.when(sid == NS - 1)
    def _():
        tbase = pl.multiple_of(cid * N + TAIL0, 8)
        pltpu.sync_copy(acc_sh.at[pl.ds(TAIL0, TAILN)], row_v.at[pl.ds(0, TAILN)])
        pltpu.sync_copy(row_v.at[pl.ds(0, TAILN)], out_hbm.at[pl.ds(tbase, TAILN)])


_deg = pl.kernel(
    _deg_body,
    out_type=jax.ShapeDtypeStruct((NC * N,), jnp.float32),
    mesh=_MESH,
    scratch_types=[
        pltpu.VMEM((CHUNK,), jnp.int32),
        pltpu.VMEM((CHUNK,), jnp.float32),
        pltpu.VMEM((ROWS_PT,), jnp.float32),
        pltpu.VMEM_SHARED((N,), jnp.float32),
    ],
)


def _agg_body(src_hbm, dst_hbm, feat_hbm, zeros_hbm, out_hbm,
              sidx_v, didx_v, rows_v, acc_sh, sem):
    cid = lax.axis_index("c")
    sid = lax.axis_index("s")
    wid = sid * NC + cid
    r0 = sid * ROWS_PT
    pltpu.sync_copy(zeros_hbm.at[pl.ds(r0, ROWS_PT)], acc_sh.at[pl.ds(r0, ROWS_PT)])

    @pl.when(sid == NS - 1)
    def _():
        pltpu.sync_copy(zeros_hbm.at[pl.ds(TAIL0, TAILN)],
                        acc_sh.at[pl.ds(TAIL0, TAILN)])

    plsc.subcore_barrier()
    ebase = wid * EPW

    def step(c, carry):
        b = ebase + c * CHUNK
        pltpu.sync_copy(src_hbm.at[pl.ds(b, CHUNK)], sidx_v)
        pltpu.sync_copy(dst_hbm.at[pl.ds(b, CHUNK)], didx_v)
        pltpu.async_copy(feat_hbm.at[sidx_v], rows_v, sem).wait()
        pltpu.sync_copy(rows_v, acc_sh.at[didx_v], add=True)
        return carry

    lax.fori_loop(0, NCH, step, 0)
    plsc.subcore_barrier()
    pltpu.sync_copy(acc_sh.at[pl.ds(r0, ROWS_PT)],
                    out_hbm.at[cid, pl.ds(r0, ROWS_PT)])

    @pl.when(sid == NS - 1)
    def _():
        pltpu.sync_copy(acc_sh.at[pl.ds(TAIL0, TAILN)],
                        out_hbm.at[cid, pl.ds(TAIL0, TAILN)])


_agg = pl.kernel(
    _agg_body,
    out_type=jax.ShapeDtypeStruct((NC, N, C), jnp.float32),
    mesh=_MESH,
    scratch_types=[
        pltpu.VMEM((CHUNK,), jnp.int32),
        pltpu.VMEM((CHUNK,), jnp.int32),
        pltpu.VMEM((CHUNK, C), jnp.float32),
        pltpu.VMEM_SHARED((N, C), jnp.float32),
        pltpu.SemaphoreType.DMA,
    ],
)


BR = 1000


def _pre_body(x_ref, w_ref, deg_ref, yp_ref, dinv_ref):
    deg = deg_ref[0] + deg_ref[1]
    dinv = lax.rsqrt(deg + 1.0)
    y = jnp.dot(x_ref[...], w_ref[...], preferred_element_type=jnp.float32)
    yp_ref[...] = dinv * y
    dinv_ref[...] = dinv


def _mid_body(acc_ref, yp_ref, dinv_ref, b_ref, hp_ref):
    s = acc_ref[0] + acc_ref[1] + yp_ref[...]
    h = jnp.maximum(dinv_ref[...] * s + b_ref[...], 0.0)
    hp_ref[...] = dinv_ref[...] * h


def _post_body(acc_ref, hp_ref, dinv_ref, w_ref, b_ref, out_ref):
    g = dinv_ref[...] * (acc_ref[0] + acc_ref[1] + hp_ref[...])
    out_ref[...] = jnp.dot(g, w_ref[...],
                           preferred_element_type=jnp.float32) + b_ref[...]


def _tc_pre(x, W1, deg):
    return pl.pallas_call(
        _pre_body,
        grid=(N // BR,),
        in_specs=[
            pl.BlockSpec((BR, C), lambda i: (i, 0)),
            pl.BlockSpec((C, C), lambda i: (0, 0)),
            pl.BlockSpec((NC, BR, 1), lambda i: (0, i, 0)),
        ],
        out_specs=[
            pl.BlockSpec((BR, C), lambda i: (i, 0)),
            pl.BlockSpec((BR, 1), lambda i: (i, 0)),
        ],
        out_shape=[
            jax.ShapeDtypeStruct((N, C), jnp.float32),
            jax.ShapeDtypeStruct((N, 1), jnp.float32),
        ],
    )(x, W1, deg)


def _tc_mid(acc, yp, dinv, b1):
    return pl.pallas_call(
        _mid_body,
        grid=(N // BR,),
        in_specs=[
            pl.BlockSpec((NC, BR, C), lambda i: (0, i, 0)),
            pl.BlockSpec((BR, C), lambda i: (i, 0)),
            pl.BlockSpec((BR, 1), lambda i: (i, 0)),
            pl.BlockSpec((1, C), lambda i: (0, 0)),
        ],
        out_specs=pl.BlockSpec((BR, C), lambda i: (i, 0)),
        out_shape=jax.ShapeDtypeStruct((N, C), jnp.float32),
    )(acc, yp, dinv, b1)


def _tc_post(acc, hp, dinv, Wcat, bcat):
    return pl.pallas_call(
        _post_body,
        grid=(N // BR,),
        in_specs=[
            pl.BlockSpec((NC, BR, C), lambda i: (0, i, 0)),
            pl.BlockSpec((BR, C), lambda i: (i, 0)),
            pl.BlockSpec((BR, 1), lambda i: (i, 0)),
            pl.BlockSpec((C, C), lambda i: (0, 0)),
            pl.BlockSpec((1, C), lambda i: (0, 0)),
        ],
        out_specs=pl.BlockSpec((BR, C), lambda i: (i, 0)),
        out_shape=jax.ShapeDtypeStruct((N, C), jnp.float32),
    )(acc, hp, dinv, Wcat, bcat)


def kernel(x, edge_index, W1, b1, Wmu, bmu, Wlog, blog):
    src = edge_index[0]
    dst = edge_index[1]
    zeros = jnp.zeros((N, C), jnp.float32)
    ones = jnp.ones((CHUNK,), jnp.float32)

    deg = _deg(dst, ones).reshape(NC, N, 1)
    yp, dinv = _tc_pre(x, W1, deg)
    acc1 = _agg(src, dst, yp, zeros)
    hp = _tc_mid(acc1, yp, dinv, b1.reshape(1, C))
    acc2 = _agg(src, dst, hp, zeros)
    Wcat = jnp.concatenate([Wmu, Wlog], axis=1)
    bcat = jnp.concatenate([bmu, blog]).reshape(1, C)
    out = _tc_post(acc2, hp, dinv, Wcat, bcat)
    return out[:, :OUT], out[:, OUT:]


# speedup vs baseline: 18.1149x; 1.1486x over previous
import functools

import jax
import jax.numpy as jnp
from jax import lax
from jax.experimental import pallas as pl
from jax.experimental.pallas import tpu as pltpu
from jax.experimental.pallas import tpu_sc as plsc

N = 10000
E = 320000
C = 128
OUT = 64
DEGW = 16

NC, NS = 2, 16
NW = NC * NS
EPW = E // NW
CHUNK = 80
NCH = EPW // CHUNK
ROWS_PT = 624
TAIL0 = NS * ROWS_PT
TAILN = N - TAIL0

_MESH = plsc.VectorSubcoreMesh(core_axis_name="c", subcore_axis_name="s")


def _deg_body(dst_hbm, ones_hbm, out_hbm, idx_v, ones_v, row_v, acc_sh):
    cid = lax.axis_index("c")
    sid = lax.axis_index("s")
    wid = sid * NC + cid
    r0 = sid * ROWS_PT
    pltpu.sync_copy(ones_hbm, ones_v)

    def zstep(i, carry):
        row_v[pl.ds(i * 16, 16)] = jnp.zeros((16,), jnp.float32)
        return carry

    lax.fori_loop(0, ROWS_PT // 16, zstep, 0)
    pltpu.sync_copy(row_v, acc_sh.at[pl.ds(r0, ROWS_PT)])

    @pl.when(sid == NS - 1)
    def _():
        pltpu.sync_copy(row_v.at[pl.ds(0, TAILN)], acc_sh.at[pl.ds(TAIL0, TAILN)])

    plsc.subcore_barrier()
    ebase = wid * EPW

    def step(c, carry):
        b = ebase + c * CHUNK
        pltpu.sync_copy(dst_hbm.at[pl.ds(b, CHUNK)], idx_v)
        pltpu.sync_copy(ones_v, acc_sh.at[idx_v], add=True)
        return carry

    lax.fori_loop(0, NCH, step, 0)
    plsc.subcore_barrier()
    obase = pl.multiple_of(cid * N + r0, 8)
    pltpu.sync_copy(acc_sh.at[pl.ds(r0, ROWS_PT)], row_v)
    pltpu.sync_copy(row_v, out_hbm.at[pl.ds(obase, ROWS_PT)])

    @pl.when(sid == NS - 1)
    def _():
        tbase = pl.multiple_of(cid * N + TAIL0, 8)
        pltpu.sync_copy(acc_sh.at[pl.ds(TAIL0, TAILN)], row_v.at[pl.ds(0, TAILN)])
        pltpu.sync_copy(row_v.at[pl.ds(0, TAILN)], out_hbm.at[pl.ds(tbase, TAILN)])


_deg = pl.kernel(
    _deg_body,
    out_type=jax.ShapeDtypeStruct((NC * N,), jnp.float32),
    mesh=_MESH,
    scratch_types=[
        pltpu.VMEM((CHUNK,), jnp.int32),
        pltpu.VMEM((CHUNK,), jnp.float32),
        pltpu.VMEM((ROWS_PT,), jnp.float32),
        pltpu.VMEM_SHARED((N,), jnp.float32),
    ],
)


def _agg_body(src_hbm, dst_hbm, feat_hbm, zeros_hbm, out_hbm,
              sidx_a, didx_a, rows_a, sidx_b, didx_b, rows_b,
              acc_sh, gsem_a, ssem_a, gsem_b, ssem_b):
    cid = lax.axis_index("c")
    sid = lax.axis_index("s")
    wid = sid * NC + cid
    r0 = sid * ROWS_PT
    pltpu.sync_copy(zeros_hbm.at[pl.ds(r0, ROWS_PT)], acc_sh.at[pl.ds(r0, ROWS_PT)])

    @pl.when(sid == NS - 1)
    def _():
        pltpu.sync_copy(zeros_hbm.at[pl.ds(TAIL0, TAILN)],
                        acc_sh.at[pl.ds(TAIL0, TAILN)])

    plsc.subcore_barrier()
    ebase = wid * EPW

    def chunk(c, sidx, didx, rows, gsem, ssem, drain):
        b = ebase + c * CHUNK
        if drain:
            pltpu.make_async_copy(rows, acc_sh.at[didx], ssem).wait()
        pltpu.sync_copy(src_hbm.at[pl.ds(b, CHUNK)], sidx)
        pltpu.sync_copy(dst_hbm.at[pl.ds(b, CHUNK)], didx)
        pltpu.async_copy(feat_hbm.at[sidx], rows, gsem).wait()
        pltpu.async_copy(rows, acc_sh.at[didx], ssem, add=True)

    chunk(0, sidx_a, didx_a, rows_a, gsem_a, ssem_a, drain=False)
    chunk(1, sidx_b, didx_b, rows_b, gsem_b, ssem_b, drain=False)

    def step(i, carry):
        chunk(2 * i, sidx_a, didx_a, rows_a, gsem_a, ssem_a, drain=True)
        chunk(2 * i + 1, sidx_b, didx_b, rows_b, gsem_b, ssem_b, drain=True)
        return carry

    lax.fori_loop(1, NCH // 2, step, 0)
    chunk(NCH - 1, sidx_a, didx_a, rows_a, gsem_a, ssem_a, drain=True)
    pltpu.make_async_copy(rows_a, acc_sh.at[didx_a], ssem_a).wait()
    pltpu.make_async_copy(rows_b, acc_sh.at[didx_b], ssem_b).wait()
    plsc.subcore_barrier()
    pltpu.sync_copy(acc_sh.at[pl.ds(r0, ROWS_PT)],
                    out_hbm.at[cid, pl.ds(r0, ROWS_PT)])

    @pl.when(sid == NS - 1)
    def _():
        pltpu.sync_copy(acc_sh.at[pl.ds(TAIL0, TAILN)],
                        out_hbm.at[cid, pl.ds(TAIL0, TAILN)])


_agg = pl.kernel(
    _agg_body,
    out_type=jax.ShapeDtypeStruct((NC, N, C), jnp.float32),
    mesh=_MESH,
    scratch_types=[
        pltpu.VMEM((CHUNK,), jnp.int32),
        pltpu.VMEM((CHUNK,), jnp.int32),
        pltpu.VMEM((CHUNK, C), jnp.float32),
        pltpu.VMEM((CHUNK,), jnp.int32),
        pltpu.VMEM((CHUNK,), jnp.int32),
        pltpu.VMEM((CHUNK, C), jnp.float32),
        pltpu.VMEM_SHARED((N, C), jnp.float32),
        pltpu.SemaphoreType.DMA,
        pltpu.SemaphoreType.DMA,
        pltpu.SemaphoreType.DMA,
        pltpu.SemaphoreType.DMA,
    ],
)


BR = 1000


def _pre_body(x_ref, w_ref, deg_ref, yp_ref, dinv_ref):
    deg = deg_ref[0] + deg_ref[1]
    dinv = lax.rsqrt(deg + 1.0)
    y = jnp.dot(x_ref[...], w_ref[...], preferred_element_type=jnp.float32)
    yp_ref[...] = dinv * y
    dinv_ref[...] = dinv


def _mid_body(acc_ref, yp_ref, dinv_ref, b_ref, hp_ref):
    s = acc_ref[0] + acc_ref[1] + yp_ref[...]
    h = jnp.maximum(dinv_ref[...] * s + b_ref[...], 0.0)
    hp_ref[...] = dinv_ref[...] * h


def _post_body(acc_ref, hp_ref, dinv_ref, w_ref, b_ref, out_ref):
    g = dinv_ref[...] * (acc_ref[0] + acc_ref[1] + hp_ref[...])
    out_ref[...] = jnp.dot(g, w_ref[...],
                           preferred_element_type=jnp.float32) + b_ref[...]


def _tc_pre(x, W1, deg):
    return pl.pallas_call(
        _pre_body,
        grid=(N // BR,),
        in_specs=[
            pl.BlockSpec((BR, C), lambda i: (i, 0)),
            pl.BlockSpec((C, C), lambda i: (0, 0)),
            pl.BlockSpec((NC, BR, 1), lambda i: (0, i, 0)),
        ],
        out_specs=[
            pl.BlockSpec((BR, C), lambda i: (i, 0)),
            pl.BlockSpec((BR, 1), lambda i: (i, 0)),
        ],
        out_shape=[
            jax.ShapeDtypeStruct((N, C), jnp.float32),
            jax.ShapeDtypeStruct((N, 1), jnp.float32),
        ],
    )(x, W1, deg)


def _tc_mid(acc, yp, dinv, b1):
    return pl.pallas_call(
        _mid_body,
        grid=(N // BR,),
        in_specs=[
            pl.BlockSpec((NC, BR, C), lambda i: (0, i, 0)),
            pl.BlockSpec((BR, C), lambda i: (i, 0)),
            pl.BlockSpec((BR, 1), lambda i: (i, 0)),
            pl.BlockSpec((1, C), lambda i: (0, 0)),
        ],
        out_specs=pl.BlockSpec((BR, C), lambda i: (i, 0)),
        out_shape=jax.ShapeDtypeStruct((N, C), jnp.float32),
    )(acc, yp, dinv, b1)


def _tc_post(acc, hp, dinv, Wcat, bcat):
    return pl.pallas_call(
        _post_body,
        grid=(N // BR,),
        in_specs=[
            pl.BlockSpec((NC, BR, C), lambda i: (0, i, 0)),
            pl.BlockSpec((BR, C), lambda i: (i, 0)),
            pl.BlockSpec((BR, 1), lambda i: (i, 0)),
            pl.BlockSpec((C, C), lambda i: (0, 0)),
            pl.BlockSpec((1, C), lambda i: (0, 0)),
        ],
        out_specs=pl.BlockSpec((BR, C), lambda i: (i, 0)),
        out_shape=jax.ShapeDtypeStruct((N, C), jnp.float32),
    )(acc, hp, dinv, Wcat, bcat)


def kernel(x, edge_index, W1, b1, Wmu, bmu, Wlog, blog):
    src = edge_index[0]
    dst = edge_index[1]
    zeros = jnp.zeros((N, C), jnp.float32)
    ones = jnp.ones((CHUNK,), jnp.float32)

    deg = _deg(dst, ones).reshape(NC, N, 1)
    yp, dinv = _tc_pre(x, W1, deg)
    acc1 = _agg(src, dst, yp, zeros)
    hp = _tc_mid(acc1, yp, dinv, b1.reshape(1, C))
    acc2 = _agg(src, dst, hp, zeros)
    Wcat = jnp.concatenate([Wmu, Wlog], axis=1)
    bcat = jnp.concatenate([bmu, blog]).reshape(1, C)
    out = _tc_post(acc2, hp, dinv, Wcat, bcat)
    return out[:, :OUT], out[:, OUT:]


# speedup vs baseline: 38.5956x; 2.1306x over previous
import functools

import jax
import jax.numpy as jnp
from jax import lax
from jax.experimental import pallas as pl
from jax.experimental.pallas import tpu as pltpu
from jax.experimental.pallas import tpu_sc as plsc

N = 10000
E = 320000
C = 128
OUT = 64
DEGW = 16

NC, NS = 2, 16
NW = NC * NS
EPW = E // NW
CHUNK = 80
NCH = EPW // CHUNK
ROWS_PT = 624
TAIL0 = NS * ROWS_PT
TAILN = N - TAIL0

_MESH = plsc.VectorSubcoreMesh(core_axis_name="c", subcore_axis_name="s")


BUFS = 5


def _deg_body(dst_hbm, ones_hbm, out_hbm, *sc):
    didx = sc[0:BUFS]
    ones_v, row_v, acc_sh = sc[BUFS], sc[BUFS + 1], sc[BUFS + 2]
    isem = sc[BUFS + 3:2 * BUFS + 3]
    ssem = sc[2 * BUFS + 3:3 * BUFS + 3]
    cid = lax.axis_index("c")
    sid = lax.axis_index("s")
    wid = sid * NC + cid
    r0 = sid * ROWS_PT
    pltpu.sync_copy(ones_hbm, ones_v)

    def zstep(i, carry):
        row_v[pl.ds(i * 16, 16)] = jnp.zeros((16,), jnp.float32)
        return carry

    lax.fori_loop(0, ROWS_PT // 16, zstep, 0)
    pltpu.sync_copy(row_v, acc_sh.at[pl.ds(r0, ROWS_PT)])

    @pl.when(sid == NS - 1)
    def _():
        pltpu.sync_copy(row_v.at[pl.ds(0, TAILN)], acc_sh.at[pl.ds(TAIL0, TAILN)])

    plsc.subcore_barrier()
    ebase = wid * EPW

    def stage(c, p, drain):
        b = ebase + c * CHUNK
        if drain:
            pltpu.make_async_copy(ones_v, acc_sh.at[didx[p]], ssem[p]).wait()
        pltpu.async_copy(dst_hbm.at[pl.ds(b, CHUNK)], didx[p], isem[p])

    def scat(p):
        pltpu.make_async_copy(dst_hbm.at[pl.ds(0, CHUNK)], didx[p], isem[p]).wait()
        pltpu.async_copy(ones_v, acc_sh.at[didx[p]], ssem[p], add=True)

    for c in range(BUFS):
        stage(c, c, drain=False)
        if c >= 2:
            scat(c - 2)

    def step(i, carry):
        for p in range(BUFS):
            c = BUFS * (i + 1) + p
            stage(c, p, drain=True)
            scat((p - 2) % BUFS)
        return carry

    lax.fori_loop(0, NCH // BUFS - 1, step, 0)
    scat((NCH - 2) % BUFS)
    scat((NCH - 1) % BUFS)
    for p in range(BUFS):
        pltpu.make_async_copy(ones_v, acc_sh.at[didx[p]], ssem[p]).wait()
    plsc.subcore_barrier()
    obase = pl.multiple_of(cid * N + r0, 8)
    pltpu.sync_copy(acc_sh.at[pl.ds(r0, ROWS_PT)], row_v)
    pltpu.sync_copy(row_v, out_hbm.at[pl.ds(obase, ROWS_PT)])

    @pl.when(sid == NS - 1)
    def _():
        tbase = pl.multiple_of(cid * N + TAIL0, 8)
        pltpu.sync_copy(acc_sh.at[pl.ds(TAIL0, TAILN)], row_v.at[pl.ds(0, TAILN)])
        pltpu.sync_copy(row_v.at[pl.ds(0, TAILN)], out_hbm.at[pl.ds(tbase, TAILN)])


_deg = pl.kernel(
    _deg_body,
    out_type=jax.ShapeDtypeStruct((NC * N,), jnp.float32),
    mesh=_MESH,
    scratch_types=(
        [pltpu.VMEM((CHUNK,), jnp.int32)] * BUFS
        + [
            pltpu.VMEM((CHUNK,), jnp.float32),
            pltpu.VMEM((ROWS_PT,), jnp.float32),
            pltpu.VMEM_SHARED((N,), jnp.float32),
        ]
        + [pltpu.SemaphoreType.DMA] * (2 * BUFS)
    ),
)


ABUFS = 4


def _agg_body(src_hbm, dst_hbm, feat_hbm, zeros_hbm, out_hbm, *sc):
    sidx = sc[0:ABUFS]
    didx = sc[ABUFS:2 * ABUFS]
    rows = sc[2 * ABUFS:3 * ABUFS]
    acc_sh = sc[3 * ABUFS]
    isem = sc[3 * ABUFS + 1:4 * ABUFS + 1]
    gsem = sc[4 * ABUFS + 1:5 * ABUFS + 1]
    ssem = sc[5 * ABUFS + 1:6 * ABUFS + 1]
    cid = lax.axis_index("c")
    sid = lax.axis_index("s")
    wid = sid * NC + cid
    r0 = sid * ROWS_PT
    pltpu.sync_copy(zeros_hbm.at[pl.ds(r0, ROWS_PT)], acc_sh.at[pl.ds(r0, ROWS_PT)])

    @pl.when(sid == NS - 1)
    def _():
        pltpu.sync_copy(zeros_hbm.at[pl.ds(TAIL0, TAILN)],
                        acc_sh.at[pl.ds(TAIL0, TAILN)])

    plsc.subcore_barrier()
    ebase = wid * EPW

    def stage(c, p, drain):
        b = ebase + c * CHUNK
        if drain:
            pltpu.make_async_copy(rows[p], acc_sh.at[didx[p]], ssem[p]).wait()
        pltpu.async_copy(src_hbm.at[pl.ds(b, CHUNK)], sidx[p], isem[p])
        pltpu.async_copy(dst_hbm.at[pl.ds(b, CHUNK)], didx[p], isem[p])

    def gath(p):
        pltpu.make_async_copy(src_hbm.at[pl.ds(0, CHUNK)], sidx[p], isem[p]).wait()
        pltpu.make_async_copy(dst_hbm.at[pl.ds(0, CHUNK)], didx[p], isem[p]).wait()
        pltpu.async_copy(feat_hbm.at[sidx[p]], rows[p], gsem[p])

    def scat(p):
        pltpu.make_async_copy(feat_hbm.at[sidx[p]], rows[p], gsem[p]).wait()
        pltpu.async_copy(rows[p], acc_sh.at[didx[p]], ssem[p], add=True)

    PEEL = 5
    for c in range(PEEL):
        stage(c, c % ABUFS, drain=(c >= ABUFS))
        if c >= 1:
            gath((c - 1) % ABUFS)
        if c >= 2:
            scat((c - 2) % ABUFS)

    def step(i, carry):
        c0 = PEEL + ABUFS * i
        for j in range(ABUFS):
            stage(c0 + j, (PEEL + j) % ABUFS, drain=True)
            gath((PEEL + j - 1) % ABUFS)
            scat((PEEL + j - 2) % ABUFS)
        return carry

    lax.fori_loop(0, (NCH - PEEL) // ABUFS, step, 0)
    gath((NCH - 1) % ABUFS)
    scat((NCH - 2) % ABUFS)
    scat((NCH - 1) % ABUFS)
    for p in range(ABUFS):
        pltpu.make_async_copy(rows[p], acc_sh.at[didx[p]], ssem[p]).wait()
    plsc.subcore_barrier()
    pltpu.sync_copy(acc_sh.at[pl.ds(r0, ROWS_PT)],
                    out_hbm.at[cid, pl.ds(r0, ROWS_PT)])

    @pl.when(sid == NS - 1)
    def _():
        pltpu.sync_copy(acc_sh.at[pl.ds(TAIL0, TAILN)],
                        out_hbm.at[cid, pl.ds(TAIL0, TAILN)])


_agg = pl.kernel(
    _agg_body,
    out_type=jax.ShapeDtypeStruct((NC, N, C), jnp.float32),
    mesh=_MESH,
    scratch_types=(
        [pltpu.VMEM((CHUNK,), jnp.int32)] * (2 * ABUFS)
        + [pltpu.VMEM((CHUNK, C), jnp.float32)] * ABUFS
        + [pltpu.VMEM_SHARED((N, C), jnp.float32)]
        + [pltpu.SemaphoreType.DMA] * (3 * ABUFS)
    ),
)


BR = 1000


def _pre_body(x_ref, w_ref, deg_ref, yp_ref, dinv_ref):
    deg = deg_ref[0] + deg_ref[1]
    dinv = lax.rsqrt(deg + 1.0)
    y = jnp.dot(x_ref[...], w_ref[...], preferred_element_type=jnp.float32)
    yp_ref[...] = dinv * y
    dinv_ref[...] = dinv


def _mid_body(acc_ref, yp_ref, dinv_ref, b_ref, hp_ref):
    s = acc_ref[0] + acc_ref[1] + yp_ref[...]
    h = jnp.maximum(dinv_ref[...] * s + b_ref[...], 0.0)
    hp_ref[...] = dinv_ref[...] * h


def _post_body(acc_ref, hp_ref, dinv_ref, w_ref, b_ref, out_ref):
    g = dinv_ref[...] * (acc_ref[0] + acc_ref[1] + hp_ref[...])
    out_ref[...] = jnp.dot(g, w_ref[...],
                           preferred_element_type=jnp.float32) + b_ref[...]


def _tc_pre(x, W1, deg):
    return pl.pallas_call(
        _pre_body,
        grid=(N // BR,),
        in_specs=[
            pl.BlockSpec((BR, C), lambda i: (i, 0)),
            pl.BlockSpec((C, C), lambda i: (0, 0)),
            pl.BlockSpec((NC, BR, 1), lambda i: (0, i, 0)),
        ],
        out_specs=[
            pl.BlockSpec((BR, C), lambda i: (i, 0)),
            pl.BlockSpec((BR, 1), lambda i: (i, 0)),
        ],
        out_shape=[
            jax.ShapeDtypeStruct((N, C), jnp.float32),
            jax.ShapeDtypeStruct((N, 1), jnp.float32),
        ],
    )(x, W1, deg)


def _tc_mid(acc, yp, dinv, b1):
    return pl.pallas_call(
        _mid_body,
        grid=(N // BR,),
        in_specs=[
            pl.BlockSpec((NC, BR, C), lambda i: (0, i, 0)),
            pl.BlockSpec((BR, C), lambda i: (i, 0)),
            pl.BlockSpec((BR, 1), lambda i: (i, 0)),
            pl.BlockSpec((1, C), lambda i: (0, 0)),
        ],
        out_specs=pl.BlockSpec((BR, C), lambda i: (i, 0)),
        out_shape=jax.ShapeDtypeStruct((N, C), jnp.float32),
    )(acc, yp, dinv, b1)


def _tc_post(acc, hp, dinv, Wcat, bcat):
    return pl.pallas_call(
        _post_body,
        grid=(N // BR,),
        in_specs=[
            pl.BlockSpec((NC, BR, C), lambda i: (0, i, 0)),
            pl.BlockSpec((BR, C), lambda i: (i, 0)),
            pl.BlockSpec((BR, 1), lambda i: (i, 0)),
            pl.BlockSpec((C, C), lambda i: (0, 0)),
            pl.BlockSpec((1, C), lambda i: (0, 0)),
        ],
        out_specs=pl.BlockSpec((BR, C), lambda i: (i, 0)),
        out_shape=jax.ShapeDtypeStruct((N, C), jnp.float32),
    )(acc, hp, dinv, Wcat, bcat)


def kernel(x, edge_index, W1, b1, Wmu, bmu, Wlog, blog):
    src = edge_index[0]
    dst = edge_index[1]
    zeros = jnp.zeros((N, C), jnp.float32)
    ones = jnp.ones((CHUNK,), jnp.float32)

    deg = _deg(dst, ones).reshape(NC, N, 1)
    yp, dinv = _tc_pre(x, W1, deg)
    acc1 = _agg(src, dst, yp, zeros)
    hp = _tc_mid(acc1, yp, dinv, b1.reshape(1, C))
    acc2 = _agg(src, dst, hp, zeros)
    Wcat = jnp.concatenate([Wmu, Wlog], axis=1)
    bcat = jnp.concatenate([bmu, blog]).reshape(1, C)
    out = _tc_post(acc2, hp, dinv, Wcat, bcat)
    return out[:, :OUT], out[:, OUT:]


# speedup vs baseline: 41.7119x; 1.0807x over previous
import jax
import jax.numpy as jnp
from jax import lax
from jax.experimental import pallas as pl
from jax.experimental.pallas import tpu as pltpu
from jax.experimental.pallas import tpu_sc as plsc

N = 10000
E = 320000
C = 128
OUT = 64

NC, NS = 2, 16
NW = NC * NS
EPW = E // NW
CHUNK = 80
NCH = EPW // CHUNK
ROWS_PT = 624
TAIL0 = NS * ROWS_PT
TAILN = N - TAIL0

_MESH = plsc.VectorSubcoreMesh(core_axis_name="c", subcore_axis_name="s")


DBUFS = 5


def _deg_body(edges_hbm, out_hbm, *sc):
    didx = sc[0:DBUFS]
    ones_v, row_v, acc_sh = sc[DBUFS:DBUFS + 3]
    isem = sc[DBUFS + 3:2 * DBUFS + 3]
    ssem = sc[2 * DBUFS + 3:3 * DBUFS + 3]
    cid = lax.axis_index("c")
    sid = lax.axis_index("s")
    wid = sid * NC + cid
    r0 = sid * ROWS_PT
    for j in range(CHUNK // 16):
        ones_v[pl.ds(16 * j, 16)] = jnp.ones((16,), jnp.float32)

    def zstep(i, carry):
        row_v[pl.ds(i * 16, 16)] = jnp.zeros((16,), jnp.float32)
        return carry

    lax.fori_loop(0, ROWS_PT // 16, zstep, 0)
    pltpu.sync_copy(row_v, acc_sh.at[pl.ds(r0, ROWS_PT)])

    @pl.when(sid == NS - 1)
    def _():
        pltpu.sync_copy(row_v.at[pl.ds(0, TAILN)], acc_sh.at[pl.ds(TAIL0, TAILN)])

    plsc.subcore_barrier()
    ebase = E + wid * EPW

    def stage(c, p, drain):
        b = ebase + c * CHUNK
        if drain:
            pltpu.make_async_copy(ones_v, acc_sh.at[didx[p]], ssem[p]).wait()
        pltpu.async_copy(edges_hbm.at[pl.ds(b, CHUNK)], didx[p], isem[p])

    def scat(p):
        pltpu.make_async_copy(edges_hbm.at[pl.ds(0, CHUNK)], didx[p], isem[p]).wait()
        pltpu.async_copy(ones_v, acc_sh.at[didx[p]], ssem[p], add=True)

    for c in range(DBUFS):
        stage(c, c, drain=False)
        if c >= 2:
            scat(c - 2)

    def step(i, carry):
        for p in range(DBUFS):
            stage(DBUFS * (i + 1) + p, p, drain=True)
            scat((p - 2) % DBUFS)
        return carry

    lax.fori_loop(0, NCH // DBUFS - 1, step, 0)
    scat((NCH - 2) % DBUFS)
    scat((NCH - 1) % DBUFS)
    for p in range(DBUFS):
        pltpu.make_async_copy(ones_v, acc_sh.at[didx[p]], ssem[p]).wait()
    plsc.subcore_barrier()
    obase = pl.multiple_of(cid * N + r0, 8)
    pltpu.sync_copy(acc_sh.at[pl.ds(r0, ROWS_PT)], row_v)
    pltpu.sync_copy(row_v, out_hbm.at[pl.ds(obase, ROWS_PT)])

    @pl.when(sid == NS - 1)
    def _():
        tbase = pl.multiple_of(cid * N + TAIL0, 8)
        pltpu.sync_copy(acc_sh.at[pl.ds(TAIL0, TAILN)], row_v.at[pl.ds(0, TAILN)])
        pltpu.sync_copy(row_v.at[pl.ds(0, TAILN)], out_hbm.at[pl.ds(tbase, TAILN)])


_deg = pl.kernel(
    _deg_body,
    out_type=jax.ShapeDtypeStruct((NC * N,), jnp.float32),
    mesh=_MESH,
    scratch_types=(
        [pltpu.VMEM((CHUNK,), jnp.int32)] * DBUFS
        + [
            pltpu.VMEM((CHUNK,), jnp.float32),
            pltpu.VMEM((ROWS_PT,), jnp.float32),
            pltpu.VMEM_SHARED((N,), jnp.float32),
        ]
        + [pltpu.SemaphoreType.DMA] * (2 * DBUFS)
    ),
)


ABUFS = 4


def _agg_body(edges_hbm, feat_hbm, out_hbm, *sc):
    sidx = sc[0:ABUFS]
    didx = sc[ABUFS:2 * ABUFS]
    rows = sc[2 * ABUFS:3 * ABUFS]
    acc_sh = sc[3 * ABUFS]
    isem = sc[3 * ABUFS + 1:4 * ABUFS + 1]
    gsem = sc[4 * ABUFS + 1:5 * ABUFS + 1]
    ssem = sc[5 * ABUFS + 1:6 * ABUFS + 1]
    cid = lax.axis_index("c")
    sid = lax.axis_index("s")
    wid = sid * NC + cid
    r0 = sid * ROWS_PT

    def zrow(r, carry):
        for j in range(8):
            rows[0][r, pl.ds(16 * j, 16)] = jnp.zeros((16,), jnp.float32)
        return carry

    lax.fori_loop(0, CHUNK, zrow, 0)
    for k in range(7):
        pltpu.sync_copy(rows[0], acc_sh.at[pl.ds(r0 + k * CHUNK, CHUNK)])
    pltpu.sync_copy(rows[0].at[pl.ds(0, 64)], acc_sh.at[pl.ds(r0 + 560, 64)])

    @pl.when(sid == NS - 1)
    def _():
        pltpu.sync_copy(rows[0].at[pl.ds(0, TAILN)], acc_sh.at[pl.ds(TAIL0, TAILN)])

    plsc.subcore_barrier()
    ebase = wid * EPW

    def stage(c, p, drain):
        b = ebase + c * CHUNK
        if drain:
            pltpu.make_async_copy(rows[p], acc_sh.at[didx[p]], ssem[p]).wait()
        pltpu.async_copy(edges_hbm.at[pl.ds(b, CHUNK)], sidx[p], isem[p])
        pltpu.async_copy(edges_hbm.at[pl.ds(E + b, CHUNK)], didx[p], isem[p])

    def gath(p):
        pltpu.make_async_copy(edges_hbm.at[pl.ds(0, CHUNK)], sidx[p], isem[p]).wait()
        pltpu.make_async_copy(edges_hbm.at[pl.ds(0, CHUNK)], didx[p], isem[p]).wait()
        pltpu.async_copy(feat_hbm.at[sidx[p]], rows[p], gsem[p])

    def scat(p):
        pltpu.make_async_copy(feat_hbm.at[sidx[p]], rows[p], gsem[p]).wait()
        pltpu.async_copy(rows[p], acc_sh.at[didx[p]], ssem[p], add=True)

    PEEL = 5
    for c in range(PEEL):
        stage(c, c % ABUFS, drain=(c >= ABUFS))
        if c >= 1:
            gath((c - 1) % ABUFS)
        if c >= 2:
            scat((c - 2) % ABUFS)

    def step(i, carry):
        c0 = PEEL + ABUFS * i
        for j in range(ABUFS):
            stage(c0 + j, (PEEL + j) % ABUFS, drain=True)
            gath((PEEL + j - 1) % ABUFS)
            scat((PEEL + j - 2) % ABUFS)
        return carry

    lax.fori_loop(0, (NCH - PEEL) // ABUFS, step, 0)
    gath((NCH - 1) % ABUFS)
    scat((NCH - 2) % ABUFS)
    scat((NCH - 1) % ABUFS)
    for p in range(ABUFS):
        pltpu.make_async_copy(rows[p], acc_sh.at[didx[p]], ssem[p]).wait()
    plsc.subcore_barrier()
    pltpu.sync_copy(acc_sh.at[pl.ds(r0, ROWS_PT)],
                    out_hbm.at[cid, pl.ds(r0, ROWS_PT)])

    @pl.when(sid == NS - 1)
    def _():
        pltpu.sync_copy(acc_sh.at[pl.ds(TAIL0, TAILN)],
                        out_hbm.at[cid, pl.ds(TAIL0, TAILN)])


_agg = pl.kernel(
    _agg_body,
    out_type=jax.ShapeDtypeStruct((NC, N, C), jnp.float32),
    mesh=_MESH,
    scratch_types=(
        [pltpu.VMEM((CHUNK,), jnp.int32)] * (2 * ABUFS)
        + [pltpu.VMEM((CHUNK, C), jnp.float32)] * ABUFS
        + [pltpu.VMEM_SHARED((N, C), jnp.float32)]
        + [pltpu.SemaphoreType.DMA] * (3 * ABUFS)
    ),
)


BR = 1000


def _pre_body(x_ref, w_ref, deg_ref, yp_ref, dinv_ref):
    dinv = lax.rsqrt(deg_ref[0] + deg_ref[1] + 1.0)
    y = jnp.dot(x_ref[...], w_ref[...], preferred_element_type=jnp.float32)
    yp_ref[...] = dinv * y
    dinv_ref[...] = dinv


def _mid_body(acc_ref, yp_ref, dinv_ref, b_ref, hp_ref):
    s = acc_ref[0] + acc_ref[1] + yp_ref[...]
    h = jnp.maximum(dinv_ref[...] * s + b_ref[...], 0.0)
    hp_ref[...] = dinv_ref[...] * h


def _post_body(acc_ref, hp_ref, dinv_ref, wmu_ref, wlog_ref, bmu_ref,
               blog_ref, mu_ref, log_ref):
    g = dinv_ref[...] * (acc_ref[0] + acc_ref[1] + hp_ref[...])
    mu_ref[...] = jnp.dot(g, wmu_ref[...],
                          preferred_element_type=jnp.float32) + bmu_ref[...]
    log_ref[...] = jnp.dot(g, wlog_ref[...],
                           preferred_element_type=jnp.float32) + blog_ref[...]


def _tc_pre(x, W1, deg):
    return pl.pallas_call(
        _pre_body,
        grid=(N // BR,),
        in_specs=[
            pl.BlockSpec((BR, C), lambda i: (i, 0)),
            pl.BlockSpec((C, C), lambda i: (0, 0)),
            pl.BlockSpec((NC, BR, 1), lambda i: (0, i, 0)),
        ],
        out_specs=[
            pl.BlockSpec((BR, C), lambda i: (i, 0)),
            pl.BlockSpec((BR, 1), lambda i: (i, 0)),
        ],
        out_shape=[
            jax.ShapeDtypeStruct((N, C), jnp.float32),
            jax.ShapeDtypeStruct((N, 1), jnp.float32),
        ],
    )(x, W1, deg)


def _tc_mid(acc, yp, dinv, b1):
    return pl.pallas_call(
        _mid_body,
        grid=(N // BR,),
        in_specs=[
            pl.BlockSpec((NC, BR, C), lambda i: (0, i, 0)),
            pl.BlockSpec((BR, C), lambda i: (i, 0)),
            pl.BlockSpec((BR, 1), lambda i: (i, 0)),
            pl.BlockSpec((1, C), lambda i: (0, 0)),
        ],
        out_specs=pl.BlockSpec((BR, C), lambda i: (i, 0)),
        out_shape=jax.ShapeDtypeStruct((N, C), jnp.float32),
    )(acc, yp, dinv, b1)


def _tc_post(acc, hp, dinv, Wmu, Wlog, bmu, blog):
    return pl.pallas_call(
        _post_body,
        grid=(N // BR,),
        in_specs=[
            pl.BlockSpec((NC, BR, C), lambda i: (0, i, 0)),
            pl.BlockSpec((BR, C), lambda i: (i, 0)),
            pl.BlockSpec((BR, 1), lambda i: (i, 0)),
            pl.BlockSpec((C, OUT), lambda i: (0, 0)),
            pl.BlockSpec((C, OUT), lambda i: (0, 0)),
            pl.BlockSpec((1, OUT), lambda i: (0, 0)),
            pl.BlockSpec((1, OUT), lambda i: (0, 0)),
        ],
        out_specs=[
            pl.BlockSpec((BR, OUT), lambda i: (i, 0)),
            pl.BlockSpec((BR, OUT), lambda i: (i, 0)),
        ],
        out_shape=[
            jax.ShapeDtypeStruct((N, OUT), jnp.float32),
            jax.ShapeDtypeStruct((N, OUT), jnp.float32),
        ],
    )(acc, hp, dinv, Wmu, Wlog, bmu, blog)


def kernel(x, edge_index, W1, b1, Wmu, bmu, Wlog, blog):
    eflat = edge_index.reshape(2 * E)
    deg = _deg(eflat).reshape(NC, N, 1)
    yp, dinv = _tc_pre(x, W1, deg)
    acc1 = _agg(eflat, yp)
    hp = _tc_mid(acc1, yp, dinv, b1.reshape(1, C))
    acc2 = _agg(eflat, hp)
    mu, log = _tc_post(acc2, hp, dinv, Wmu, Wlog,
                       bmu.reshape(1, OUT), blog.reshape(1, OUT))
    return mu, log


# speedup vs baseline: 45.5936x; 1.0931x over previous
import jax
import jax.numpy as jnp
from jax import lax
from jax.experimental import pallas as pl
from jax.experimental.pallas import tpu as pltpu
from jax.experimental.pallas import tpu_sc as plsc

N = 10000
E = 320000
C = 128
OUT = 64

NC, NS = 2, 16
NW = NC * NS
EPW = E // NW
CHUNK = 80
NCH = EPW // CHUNK
ROWS_PT = 624
TAIL0 = NS * ROWS_PT
TAILN = N - TAIL0

_MESH = plsc.VectorSubcoreMesh(core_axis_name="c", subcore_axis_name="s")


DBUFS = 5


def _deg_body(edges_hbm, out_hbm, *sc):
    didx = sc[0:DBUFS]
    ones_v, row_v, acc_sh = sc[DBUFS:DBUFS + 3]
    isem = sc[DBUFS + 3:2 * DBUFS + 3]
    ssem = sc[2 * DBUFS + 3:3 * DBUFS + 3]
    cid = lax.axis_index("c")
    sid = lax.axis_index("s")
    wid = sid * NC + cid
    r0 = sid * ROWS_PT
    for j in range(CHUNK // 16):
        ones_v[pl.ds(16 * j, 16)] = jnp.ones((16,), jnp.float32)

    def zstep(i, carry):
        row_v[pl.ds(i * 16, 16)] = jnp.zeros((16,), jnp.float32)
        return carry

    lax.fori_loop(0, ROWS_PT // 16, zstep, 0)
    pltpu.sync_copy(row_v, acc_sh.at[pl.ds(r0, ROWS_PT)])

    @pl.when(sid == NS - 1)
    def _():
        pltpu.sync_copy(row_v.at[pl.ds(0, TAILN)], acc_sh.at[pl.ds(TAIL0, TAILN)])

    plsc.subcore_barrier()
    ebase = E + wid * EPW

    def stage(c, p, drain):
        b = ebase + c * CHUNK
        if drain:
            pltpu.make_async_copy(ones_v, acc_sh.at[didx[p]], ssem[p]).wait()
        pltpu.async_copy(edges_hbm.at[pl.ds(b, CHUNK)], didx[p], isem[p])

    def scat(p):
        pltpu.make_async_copy(edges_hbm.at[pl.ds(0, CHUNK)], didx[p], isem[p]).wait()
        pltpu.async_copy(ones_v, acc_sh.at[didx[p]], ssem[p], add=True)

    for c in range(DBUFS):
        stage(c, c, drain=False)
        if c >= 2:
            scat(c - 2)

    def step(i, carry):
        for p in range(DBUFS):
            stage(DBUFS * (i + 1) + p, p, drain=True)
            scat((p - 2) % DBUFS)
        return carry

    lax.fori_loop(0, NCH // DBUFS - 1, step, 0)
    scat((NCH - 2) % DBUFS)
    scat((NCH - 1) % DBUFS)
    for p in range(DBUFS):
        pltpu.make_async_copy(ones_v, acc_sh.at[didx[p]], ssem[p]).wait()
    plsc.subcore_barrier()
    obase = pl.multiple_of(cid * N + r0, 8)
    pltpu.sync_copy(acc_sh.at[pl.ds(r0, ROWS_PT)], row_v)
    pltpu.sync_copy(row_v, out_hbm.at[pl.ds(obase, ROWS_PT)])

    @pl.when(sid == NS - 1)
    def _():
        tbase = pl.multiple_of(cid * N + TAIL0, 8)
        pltpu.sync_copy(acc_sh.at[pl.ds(TAIL0, TAILN)], row_v.at[pl.ds(0, TAILN)])
        pltpu.sync_copy(row_v.at[pl.ds(0, TAILN)], out_hbm.at[pl.ds(tbase, TAILN)])


_deg = pl.kernel(
    _deg_body,
    out_type=jax.ShapeDtypeStruct((NC * N,), jnp.float32),
    mesh=_MESH,
    scratch_types=(
        [pltpu.VMEM((CHUNK,), jnp.int32)] * DBUFS
        + [
            pltpu.VMEM((CHUNK,), jnp.float32),
            pltpu.VMEM((ROWS_PT,), jnp.float32),
            pltpu.VMEM_SHARED((N,), jnp.float32),
        ]
        + [pltpu.SemaphoreType.DMA] * (2 * DBUFS)
    ),
)


ABUFS = 4


def _agg_body(edges_hbm, feat_hbm, out_hbm, *sc):
    sidx = sc[0:ABUFS]
    didx = sc[ABUFS:2 * ABUFS]
    rows = sc[2 * ABUFS:3 * ABUFS]
    acc_sh = sc[3 * ABUFS]
    isem = sc[3 * ABUFS + 1:4 * ABUFS + 1]
    gsem = sc[4 * ABUFS + 1:5 * ABUFS + 1]
    ssem = sc[5 * ABUFS + 1:6 * ABUFS + 1]
    cid = lax.axis_index("c")
    sid = lax.axis_index("s")
    wid = sid * NC + cid
    r0 = sid * ROWS_PT

    def zrow(r, carry):
        for j in range(8):
            rows[0][r, pl.ds(16 * j, 16)] = jnp.zeros((16,), jnp.float32)
        return carry

    lax.fori_loop(0, CHUNK, zrow, 0)
    for k in range(7):
        pltpu.sync_copy(rows[0], acc_sh.at[pl.ds(r0 + k * CHUNK, CHUNK)])
    pltpu.sync_copy(rows[0].at[pl.ds(0, 64)], acc_sh.at[pl.ds(r0 + 560, 64)])

    @pl.when(sid == NS - 1)
    def _():
        pltpu.sync_copy(rows[0].at[pl.ds(0, TAILN)], acc_sh.at[pl.ds(TAIL0, TAILN)])

    plsc.subcore_barrier()
    ebase = wid * EPW

    def stage(c, p, drain):
        b = ebase + c * CHUNK
        if drain:
            pltpu.make_async_copy(rows[p], acc_sh.at[didx[p]], ssem[p]).wait()
        pltpu.async_copy(edges_hbm.at[pl.ds(b, CHUNK)], sidx[p], isem[p])
        pltpu.async_copy(edges_hbm.at[pl.ds(E + b, CHUNK)], didx[p], isem[p])

    def gath(p):
        pltpu.make_async_copy(edges_hbm.at[pl.ds(0, CHUNK)], sidx[p], isem[p]).wait()
        pltpu.make_async_copy(edges_hbm.at[pl.ds(0, CHUNK)], didx[p], isem[p]).wait()
        pltpu.async_copy(feat_hbm.at[sidx[p]], rows[p], gsem[p])

    def scat(p):
        pltpu.make_async_copy(feat_hbm.at[sidx[p]], rows[p], gsem[p]).wait()
        pltpu.async_copy(rows[p], acc_sh.at[didx[p]], ssem[p], add=True)

    PEEL = 5
    for c in range(PEEL):
        stage(c, c % ABUFS, drain=(c >= ABUFS))
        if c >= 1:
            gath((c - 1) % ABUFS)
        if c >= 2:
            scat((c - 2) % ABUFS)

    def step(i, carry):
        c0 = PEEL + ABUFS * i
        for j in range(ABUFS):
            stage(c0 + j, (PEEL + j) % ABUFS, drain=True)
            gath((PEEL + j - 1) % ABUFS)
            scat((PEEL + j - 2) % ABUFS)
        return carry

    lax.fori_loop(0, (NCH - PEEL) // ABUFS, step, 0)
    gath((NCH - 1) % ABUFS)
    scat((NCH - 2) % ABUFS)
    scat((NCH - 1) % ABUFS)
    for p in range(ABUFS):
        pltpu.make_async_copy(rows[p], acc_sh.at[didx[p]], ssem[p]).wait()
    plsc.subcore_barrier()
    pltpu.sync_copy(acc_sh.at[pl.ds(r0, ROWS_PT)],
                    out_hbm.at[cid, pl.ds(r0, ROWS_PT)])

    @pl.when(sid == NS - 1)
    def _():
        pltpu.sync_copy(acc_sh.at[pl.ds(TAIL0, TAILN)],
                        out_hbm.at[cid, pl.ds(TAIL0, TAILN)])


_agg = pl.kernel(
    _agg_body,
    out_type=jax.ShapeDtypeStruct((NC, N, C), jnp.float32),
    mesh=_MESH,
    scratch_types=(
        [pltpu.VMEM((CHUNK,), jnp.int32)] * (2 * ABUFS)
        + [pltpu.VMEM((CHUNK, C), jnp.float32)] * ABUFS
        + [pltpu.VMEM_SHARED((N, C), jnp.float32)]
        + [pltpu.SemaphoreType.DMA] * (3 * ABUFS)
    ),
)


def _pre_body(x_ref, w_ref, deg_ref, yp_ref, dinv_ref):
    dvec = lax.rsqrt(deg_ref[pl.ds(0, N)] + deg_ref[pl.ds(N, N)] + 1.0)
    dinv = dvec.reshape(N, 1)
    y = jnp.dot(x_ref[...], w_ref[...], preferred_element_type=jnp.float32)
    yp_ref[...] = dinv * y
    dinv_ref[...] = dvec


def _mid_body(acc_ref, yp_ref, dinv_ref, b_ref, hp_ref):
    dinv = dinv_ref[...].reshape(N, 1)
    s = acc_ref[0] + acc_ref[1] + yp_ref[...]
    h = jnp.maximum(dinv * s + b_ref[...], 0.0)
    hp_ref[...] = dinv * h


def _post_body(acc_ref, hp_ref, dinv_ref, wmu_ref, wlog_ref, bmu_ref,
               blog_ref, mu_ref, log_ref):
    dinv = dinv_ref[...].reshape(N, 1)
    g = dinv * (acc_ref[0] + acc_ref[1] + hp_ref[...])
    mu_ref[...] = jnp.dot(g, wmu_ref[...],
                          preferred_element_type=jnp.float32) + bmu_ref[...]
    log_ref[...] = jnp.dot(g, wlog_ref[...],
                           preferred_element_type=jnp.float32) + blog_ref[...]


def _tc_pre(x, W1, deg):
    return pl.pallas_call(
        _pre_body,
        out_shape=[
            jax.ShapeDtypeStruct((N, C), jnp.float32),
            jax.ShapeDtypeStruct((N,), jnp.float32),
        ],
    )(x, W1, deg)


def _tc_mid(acc, yp, dinv, b1):
    return pl.pallas_call(
        _mid_body,
        out_shape=jax.ShapeDtypeStruct((N, C), jnp.float32),
    )(acc, yp, dinv, b1)


def _tc_post(acc, hp, dinv, Wmu, Wlog, bmu, blog):
    return pl.pallas_call(
        _post_body,
        out_shape=[
            jax.ShapeDtypeStruct((N, OUT), jnp.float32),
            jax.ShapeDtypeStruct((N, OUT), jnp.float32),
        ],
    )(acc, hp, dinv, Wmu, Wlog, bmu, blog)


def kernel(x, edge_index, W1, b1, Wmu, bmu, Wlog, blog):
    eflat = edge_index.reshape(2 * E)
    deg = _deg(eflat)
    yp, dinv = _tc_pre(x, W1, deg)
    acc1 = _agg(eflat, yp)
    hp = _tc_mid(acc1, yp, dinv, b1.reshape(1, C))
    acc2 = _agg(eflat, hp)
    mu, log = _tc_post(acc2, hp, dinv, Wmu, Wlog,
                       bmu.reshape(1, OUT), blog.reshape(1, OUT))
    return mu, log


# speedup vs baseline: 45.7738x; 1.0040x over previous
import jax
import jax.numpy as jnp
from jax import lax
from jax.experimental import pallas as pl
from jax.experimental.pallas import tpu as pltpu
from jax.experimental.pallas import tpu_sc as plsc

N = 10000
E = 320000
C = 128
OUT = 64

NC, NS = 2, 16
NW = NC * NS
EPW = E // NW
CHUNK = 80
NCH = EPW // CHUNK
ROWS_PT = 624
TAIL0 = NS * ROWS_PT
TAILN = N - TAIL0

_MESH = plsc.VectorSubcoreMesh(core_axis_name="c", subcore_axis_name="s")


DBUFS = 5


def _deg_body(edges_hbm, out_hbm, *sc):
    didx = sc[0:DBUFS]
    ones_v, row_v, acc_sh = sc[DBUFS:DBUFS + 3]
    isem = sc[DBUFS + 3:2 * DBUFS + 3]
    ssem = sc[2 * DBUFS + 3:3 * DBUFS + 3]
    cid = lax.axis_index("c")
    sid = lax.axis_index("s")
    wid = sid * NC + cid
    r0 = sid * ROWS_PT
    for j in range(CHUNK // 16):
        ones_v[pl.ds(16 * j, 16)] = jnp.ones((16,), jnp.float32)

    def zstep(i, carry):
        row_v[pl.ds(i * 16, 16)] = jnp.zeros((16,), jnp.float32)
        return carry

    lax.fori_loop(0, ROWS_PT // 16, zstep, 0)
    pltpu.sync_copy(row_v, acc_sh.at[pl.ds(r0, ROWS_PT)])

    @pl.when(sid == NS - 1)
    def _():
        pltpu.sync_copy(row_v.at[pl.ds(0, TAILN)], acc_sh.at[pl.ds(TAIL0, TAILN)])

    plsc.subcore_barrier()
    ebase = E + wid * EPW

    def stage(c, p, drain):
        b = ebase + c * CHUNK
        if drain:
            pltpu.make_async_copy(ones_v, acc_sh.at[didx[p]], ssem[p]).wait()
        pltpu.async_copy(edges_hbm.at[pl.ds(b, CHUNK)], didx[p], isem[p])

    def scat(p):
        pltpu.make_async_copy(edges_hbm.at[pl.ds(0, CHUNK)], didx[p], isem[p]).wait()
        pltpu.async_copy(ones_v, acc_sh.at[didx[p]], ssem[p], add=True)

    for c in range(DBUFS):
        stage(c, c, drain=False)
        if c >= 2:
            scat(c - 2)

    def step(i, carry):
        for p in range(DBUFS):
            stage(DBUFS * (i + 1) + p, p, drain=True)
            scat((p - 2) % DBUFS)
        return carry

    lax.fori_loop(0, NCH // DBUFS - 1, step, 0)
    scat((NCH - 2) % DBUFS)
    scat((NCH - 1) % DBUFS)
    for p in range(DBUFS):
        pltpu.make_async_copy(ones_v, acc_sh.at[didx[p]], ssem[p]).wait()
    plsc.subcore_barrier()
    obase = pl.multiple_of(cid * N + r0, 8)
    pltpu.sync_copy(acc_sh.at[pl.ds(r0, ROWS_PT)], row_v)
    pltpu.sync_copy(row_v, out_hbm.at[pl.ds(obase, ROWS_PT)])

    @pl.when(sid == NS - 1)
    def _():
        tbase = pl.multiple_of(cid * N + TAIL0, 8)
        pltpu.sync_copy(acc_sh.at[pl.ds(TAIL0, TAILN)], row_v.at[pl.ds(0, TAILN)])
        pltpu.sync_copy(row_v.at[pl.ds(0, TAILN)], out_hbm.at[pl.ds(tbase, TAILN)])


_deg = pl.kernel(
    _deg_body,
    out_type=jax.ShapeDtypeStruct((NC * N,), jnp.float32),
    mesh=_MESH,
    scratch_types=(
        [pltpu.VMEM((CHUNK,), jnp.int32)] * DBUFS
        + [
            pltpu.VMEM((CHUNK,), jnp.float32),
            pltpu.VMEM((ROWS_PT,), jnp.float32),
            pltpu.VMEM_SHARED((N,), jnp.float32),
        ]
        + [pltpu.SemaphoreType.DMA] * (2 * DBUFS)
    ),
)


ABUFS = 4


def _agg_body(edges_hbm, feat_hbm, out_hbm, *sc):
    sidx = sc[0:ABUFS]
    didx = sc[ABUFS:2 * ABUFS]
    rows = sc[2 * ABUFS:3 * ABUFS]
    acc_sh = sc[3 * ABUFS]
    isem = sc[3 * ABUFS + 1:4 * ABUFS + 1]
    gsem = sc[4 * ABUFS + 1:5 * ABUFS + 1]
    ssem = sc[5 * ABUFS + 1:6 * ABUFS + 1]
    cid = lax.axis_index("c")
    sid = lax.axis_index("s")
    wid = sid * NC + cid
    r0 = sid * ROWS_PT

    ebase = wid * EPW

    def stage(c, p, drain):
        b = ebase + c * CHUNK
        if drain:
            pltpu.make_async_copy(rows[p], acc_sh.at[didx[p]], ssem[p]).wait()
        pltpu.async_copy(edges_hbm.at[pl.ds(b, CHUNK)], sidx[p], isem[p])
        pltpu.async_copy(edges_hbm.at[pl.ds(E + b, CHUNK)], didx[p], isem[p])

    def gath(p):
        pltpu.make_async_copy(edges_hbm.at[pl.ds(0, CHUNK)], sidx[p], isem[p]).wait()
        pltpu.make_async_copy(edges_hbm.at[pl.ds(0, CHUNK)], didx[p], isem[p]).wait()
        pltpu.async_copy(feat_hbm.at[sidx[p]], rows[p], gsem[p])

    def scat(p):
        pltpu.make_async_copy(feat_hbm.at[sidx[p]], rows[p], gsem[p]).wait()
        pltpu.async_copy(rows[p], acc_sh.at[didx[p]], ssem[p], add=True)

    PEEL = 5
    stage(0, 0, drain=False)
    stage(1, 1, drain=False)
    gath(0)

    def zrow(r, carry):
        for j in range(8):
            rows[3][r, pl.ds(16 * j, 16)] = jnp.zeros((16,), jnp.float32)
        return carry

    lax.fori_loop(0, CHUNK, zrow, 0)
    for k in range(7):
        pltpu.sync_copy(rows[3], acc_sh.at[pl.ds(r0 + k * CHUNK, CHUNK)])
    pltpu.sync_copy(rows[3].at[pl.ds(0, 64)], acc_sh.at[pl.ds(r0 + 560, 64)])

    @pl.when(sid == NS - 1)
    def _():
        pltpu.sync_copy(rows[3].at[pl.ds(0, TAILN)], acc_sh.at[pl.ds(TAIL0, TAILN)])

    plsc.subcore_barrier()
    for c in range(2, PEEL):
        stage(c, c % ABUFS, drain=(c >= ABUFS))
        gath((c - 1) % ABUFS)
        scat((c - 2) % ABUFS)

    def step(i, carry):
        c0 = PEEL + ABUFS * i
        for j in range(ABUFS):
            stage(c0 + j, (PEEL + j) % ABUFS, drain=True)
            gath((PEEL + j - 1) % ABUFS)
            scat((PEEL + j - 2) % ABUFS)
        return carry

    lax.fori_loop(0, (NCH - PEEL) // ABUFS, step, 0)
    gath((NCH - 1) % ABUFS)
    scat((NCH - 2) % ABUFS)
    scat((NCH - 1) % ABUFS)
    for p in range(ABUFS):
        pltpu.make_async_copy(rows[p], acc_sh.at[didx[p]], ssem[p]).wait()
    plsc.subcore_barrier()
    pltpu.sync_copy(acc_sh.at[pl.ds(r0, ROWS_PT)],
                    out_hbm.at[cid, pl.ds(r0, ROWS_PT)])

    @pl.when(sid == NS - 1)
    def _():
        pltpu.sync_copy(acc_sh.at[pl.ds(TAIL0, TAILN)],
                        out_hbm.at[cid, pl.ds(TAIL0, TAILN)])


_agg = pl.kernel(
    _agg_body,
    out_type=jax.ShapeDtypeStruct((NC, N, C), jnp.float32),
    mesh=_MESH,
    scratch_types=(
        [pltpu.VMEM((CHUNK,), jnp.int32)] * (2 * ABUFS)
        + [pltpu.VMEM((CHUNK, C), jnp.float32)] * ABUFS
        + [pltpu.VMEM_SHARED((N, C), jnp.float32)]
        + [pltpu.SemaphoreType.DMA] * (3 * ABUFS)
    ),
)


def _mm_body(x_ref, w_ref, y_ref):
    y_ref[...] = jnp.dot(x_ref[...], w_ref[...],
                         preferred_element_type=jnp.float32)


def _pre_body(y_ref, deg_ref, yp_ref, dinv_ref):
    dvec = lax.rsqrt(deg_ref[pl.ds(0, N)] + deg_ref[pl.ds(N, N)] + 1.0)
    dinv = dvec.reshape(N, 1)
    yp_ref[...] = dinv * y_ref[...]
    dinv_ref[...] = dvec


def _mid_body(acc_ref, yp_ref, dinv_ref, b_ref, hp_ref):
    dinv = dinv_ref[...].reshape(N, 1)
    s = acc_ref[0] + acc_ref[1] + yp_ref[...]
    h = jnp.maximum(dinv * s + b_ref[...], 0.0)
    hp_ref[...] = dinv * h


def _post_body(acc_ref, hp_ref, dinv_ref, wmu_ref, wlog_ref, bmu_ref,
               blog_ref, mu_ref, log_ref):
    dinv = dinv_ref[...].reshape(N, 1)
    g = dinv * (acc_ref[0] + acc_ref[1] + hp_ref[...])
    mu_ref[...] = jnp.dot(g, wmu_ref[...],
                          preferred_element_type=jnp.float32) + bmu_ref[...]
    log_ref[...] = jnp.dot(g, wlog_ref[...],
                           preferred_element_type=jnp.float32) + blog_ref[...]


def _tc_mm(x, W1):
    return pl.pallas_call(
        _mm_body,
        out_shape=jax.ShapeDtypeStruct((N, C), jnp.float32),
    )(x, W1)


def _tc_pre(y, deg):
    return pl.pallas_call(
        _pre_body,
        out_shape=[
            jax.ShapeDtypeStruct((N, C), jnp.float32),
            jax.ShapeDtypeStruct((N,), jnp.float32),
        ],
    )(y, deg)


def _tc_mid(acc, yp, dinv, b1):
    return pl.pallas_call(
        _mid_body,
        out_shape=jax.ShapeDtypeStruct((N, C), jnp.float32),
    )(acc, yp, dinv, b1)


def _tc_post(acc, hp, dinv, Wmu, Wlog, bmu, blog):
    return pl.pallas_call(
        _post_body,
        out_shape=[
            jax.ShapeDtypeStruct((N, OUT), jnp.float32),
            jax.ShapeDtypeStruct((N, OUT), jnp.float32),
        ],
    )(acc, hp, dinv, Wmu, Wlog, bmu, blog)


def kernel(x, edge_index, W1, b1, Wmu, bmu, Wlog, blog):
    eflat = edge_index.reshape(2 * E)
    deg = _deg(eflat)
    y = _tc_mm(x, W1)
    yp, dinv = _tc_pre(y, deg)
    acc1 = _agg(eflat, yp)
    hp = _tc_mid(acc1, yp, dinv, b1.reshape(1, C))
    acc2 = _agg(eflat, hp)
    mu, log = _tc_post(acc2, hp, dinv, Wmu, Wlog,
                       bmu.reshape(1, OUT), blog.reshape(1, OUT))
    return mu, log
